# baseline (speedup 1.0000x reference)
import jax
import jax.numpy as jnp
from jax import lax
from jax.experimental import pallas as pl
from jax.experimental.pallas import tpu as pltpu
from jax.experimental.pallas import tpu_sc as plsc

B = 4
SEQ = 8192
D = 1024
NC = 2
NS = 16
LANES = 16
NW = NC * NS
SPW = SEQ // NW
CH = 4
PECH = 2 * CH
NCH = SPW // CH
NPE = SPW // PECH
XSLOTS = 4
SPP = 4
NOUTER = NCH // SPP
GROUPS = D // LANES
LOOKAHEAD = 3


def _body(x_hbm, idx_hbm, pe_hbm, out_hbm, idx_v, pe_v, x_v,
          sem_pe, sem_in, sem_out):
    wid = lax.axis_index("s") * NC + lax.axis_index("c")
    base = wid * SPW
    pltpu.sync_copy(idx_hbm.at[pl.ds(base, SPW)], idx_v)

    def pe_gather(p, pb):
        pltpu.async_copy(
            pe_hbm.at[idx_v.at[pl.ds(p * PECH, PECH)]],
            pe_v.at[pl.ds(pb * PECH, PECH)], sem_pe.at[pb])

    def wait_pe(pb):
        pltpu.make_async_copy(
            pe_hbm.at[pl.ds(0, PECH)],
            pe_v.at[pl.ds(pb * PECH, PECH)], sem_pe.at[pb]).wait()

    def in_issue(c, k):
        pltpu.async_copy(
            x_hbm.at[:, pl.ds(base + c * CH, CH)], x_v.at[k], sem_in.at[k])

    def wait_in(k):
        pltpu.make_async_copy(
            x_hbm.at[:, pl.ds(0, CH)], x_v.at[k], sem_in.at[k]).wait()

    def out_issue(c, k):
        pltpu.async_copy(
            x_v.at[k], out_hbm.at[:, pl.ds(base + c * CH, CH)], sem_out.at[k])

    def wait_out(k):
        pltpu.make_async_copy(
            x_v.at[k], out_hbm.at[:, pl.ds(0, CH)], sem_out.at[k]).wait()

    pe_gather(0, 0)
    pe_gather(1, 1)
    for t in range(LOOKAHEAD):
        in_issue(t, t % XSLOTS)

    def outer(j, carry):
        for q in range(SPP):
            pb = q // 2
            wait_in(q)
            if q % 2 == 0:
                wait_pe(pb)

            peoff = pb * PECH + (q % 2) * CH
            for bb in range(B):

                def add_rows(r, _, q=q, bb=bb, peoff=peoff):
                    for g in range(GROUPS):
                        sl = pl.ds(g * LANES, LANES)
                        plsc.addupdate(x_v.at[q, bb, r, sl],
                                       pe_v[peoff + r, sl])
                    return 0

                lax.fori_loop(0, CH, add_rows, 0)

            if q % 2 == 1:
                @pl.when(j <= NOUTER - 2)
                def _(j_=j, pb=pb):
                    pe_gather(2 * j_ + pb + 2, pb)

            out_issue(SPP * j + q, q)

            tq = q + LOOKAHEAD
            k2 = tq % XSLOTS
            j_off = tq // SPP
            jmax = (NCH - 1 - LOOKAHEAD - q) // SPP

            @pl.when(j <= jmax)
            def _(j_=j, q=q, k2=k2, j_off=j_off, tq=tq):
                if q >= XSLOTS - LOOKAHEAD:
                    wait_out(k2)
                else:
                    @pl.when(j_ >= 1)
                    def _():
                        wait_out(k2)
                in_issue(SPP * (j_ + j_off) + tq % SPP, k2)
        return carry

    lax.fori_loop(0, NOUTER, outer, 0)

    for k in range(XSLOTS):
        wait_out(k)


@jax.jit
def _pe_add(x, indices, pe):
    mesh = plsc.VectorSubcoreMesh(core_axis_name="c", subcore_axis_name="s")
    return pl.kernel(
        _body,
        out_type=jax.ShapeDtypeStruct((B, SEQ, D), jnp.float32),
        mesh=mesh,
        scratch_types=[
            pltpu.VMEM((SPW,), jnp.int32),
            pltpu.VMEM((2 * PECH, D), jnp.float32),
            pltpu.VMEM((XSLOTS, B, CH, D), jnp.float32),
            pltpu.SemaphoreType.DMA((2,)),
            pltpu.SemaphoreType.DMA((XSLOTS,)),
            pltpu.SemaphoreType.DMA((XSLOTS,)),
        ],
    )(x, indices, pe)


def kernel(x, indices, pe):
    return _pe_add(x, indices.astype(jnp.int32), pe)
